# Initial kernel scaffold; baseline (speedup 1.0000x reference)
import jax
import jax.numpy as jnp
from jax.experimental import pallas as pl


def kernel(x, edge_index, W0, b0, W1, b1, W2, b2, W3, b3, fcW, fcb):
    raise NotImplementedError("write your pallas kernel here")


# speedup vs baseline: 5.1470x; 5.1470x over previous
import functools

import jax
import jax.numpy as jnp
from jax import lax
from jax.experimental import pallas as pl
from jax.experimental.pallas import tpu as pltpu
from jax.experimental.pallas import tpu_sc as plsc

N = 10000
F = 128
CLS = 64

NC = 2
NS = 16
NW = NC * NS
KE = 128
EPW = 10240
EPAD = NW * EPW
NCHUNK = EPW // KE
NPAD = 10112
STRIPE = NPAD // NS
DEGW = 128
ZR = 64

_mesh = plsc.VectorSubcoreMesh(core_axis_name="c", subcore_axis_name="s")


def _zero_stripe(zbuf, acc, rows0, width):
    @pl.loop(0, ZR)
    def _(r):
        @pl.loop(0, width, step=16)
        def _(c):
            zbuf[r, pl.ds(c, 16)] = jnp.zeros((16,), jnp.float32)

    nfull = (STRIPE // ZR) * ZR

    @pl.loop(0, nfull, step=ZR)
    def _(r):
        pltpu.sync_copy(zbuf, acc.at[pl.ds(rows0 + r, ZR)])

    rem = STRIPE - nfull
    if rem:
        pltpu.sync_copy(zbuf.at[pl.ds(0, rem)], acc.at[pl.ds(rows0 + nfull, rem)])


@functools.partial(
    pl.kernel,
    out_type=jax.ShapeDtypeStruct((NC, NPAD, DEGW), jnp.float32),
    mesh=_mesh,
    scratch_types=[
        pltpu.VMEM((KE,), jnp.int32),
        pltpu.VMEM((KE, DEGW), jnp.float32),
        pltpu.VMEM((ZR, DEGW), jnp.float32),
        pltpu.VMEM_SHARED((NPAD, DEGW), jnp.float32),
    ],
)
def _sc_degree(dst_hbm, out_hbm, dstv, ones, zbuf, acc):
    cid = lax.axis_index("c")
    sid = lax.axis_index("s")
    wid = sid * NC + cid
    rows0 = sid * STRIPE

    @pl.loop(0, KE)
    def _(r):
        @pl.loop(0, DEGW, step=16)
        def _(c):
            ones[r, pl.ds(c, 16)] = jnp.full((16,), 1.0, jnp.float32)

    _zero_stripe(zbuf, acc, rows0, DEGW)
    plsc.subcore_barrier()

    base = wid * EPW

    @pl.loop(0, NCHUNK)
    def _(i):
        pltpu.sync_copy(dst_hbm.at[pl.ds(base + i * KE, KE)], dstv)
        pltpu.sync_copy(ones, acc.at[dstv], add=True)

    plsc.subcore_barrier()
    pltpu.sync_copy(acc.at[pl.ds(rows0, STRIPE)],
                    out_hbm.at[cid, pl.ds(rows0, STRIPE)])


@functools.partial(
    pl.kernel,
    out_type=jax.ShapeDtypeStruct((NC, NPAD, F), jnp.float32),
    mesh=_mesh,
    scratch_types=[
        pltpu.VMEM((KE,), jnp.int32),
        pltpu.VMEM((KE,), jnp.int32),
        pltpu.VMEM((KE, F), jnp.float32),
        pltpu.VMEM((ZR, F), jnp.float32),
        pltpu.VMEM_SHARED((NPAD, F), jnp.float32),
        pltpu.SemaphoreType.DMA,
    ],
)
def _sc_aggregate(hws_hbm, src_hbm, dst_hbm, out_hbm, srcv, dstv, rows, zbuf,
                  acc, sem):
    cid = lax.axis_index("c")
    sid = lax.axis_index("s")
    wid = sid * NC + cid
    rows0 = sid * STRIPE

    _zero_stripe(zbuf, acc, rows0, F)
    plsc.subcore_barrier()

    base = wid * EPW

    @pl.loop(0, NCHUNK)
    def _(i):
        off = base + i * KE
        pltpu.sync_copy(src_hbm.at[pl.ds(off, KE)], srcv)
        pltpu.sync_copy(dst_hbm.at[pl.ds(off, KE)], dstv)
        pltpu.async_copy(hws_hbm.at[srcv], rows, sem).wait()
        pltpu.sync_copy(rows, acc.at[dstv], add=True)

    plsc.subcore_barrier()
    pltpu.sync_copy(acc.at[pl.ds(rows0, STRIPE)],
                    out_hbm.at[cid, pl.ds(rows0, STRIPE)])


def _dot(a, b):
    return jnp.dot(a, b, preferred_element_type=jnp.float32,
                   precision=lax.Precision.HIGHEST)


def _tc_first_body(x_ref, w_ref, degp_ref, hws_ref, dinv_ref):
    deg = degp_ref[0] + degp_ref[1] + 1.0
    dinv = lax.rsqrt(jnp.maximum(deg, 1.0))
    hws_ref[...] = _dot(x_ref[...], w_ref[...]) * dinv
    dinv_ref[...] = dinv


def _tc_mid_first_body(p_ref, hws_ref, dinv_ref, b_ref, w_ref,
                       hmaxo_ref, hwsn_ref):
    dinv = dinv_ref[...]
    pre = p_ref[0] + p_ref[1] + hws_ref[...]
    h = jnp.maximum(pre * dinv + b_ref[...], 0.0)
    hmaxo_ref[...] = h
    hwsn_ref[...] = _dot(h, w_ref[...]) * dinv


def _tc_mid_body(p_ref, hws_ref, dinv_ref, b_ref, w_ref, hmax_ref,
                 hmaxo_ref, hwsn_ref):
    dinv = dinv_ref[...]
    pre = p_ref[0] + p_ref[1] + hws_ref[...]
    h = jnp.maximum(pre * dinv + b_ref[...], 0.0)
    hmaxo_ref[...] = jnp.maximum(hmax_ref[...], h)
    hwsn_ref[...] = _dot(h, w_ref[...]) * dinv


def _tc_last_body(p_ref, hws_ref, dinv_ref, b_ref, hmax_ref, fcw_ref, fcb_ref,
                  out_ref):
    pre = p_ref[0] + p_ref[1] + hws_ref[...]
    h = jnp.maximum(pre * dinv_ref[...] + b_ref[...], 0.0)
    hj = jnp.maximum(hmax_ref[...], h)
    logits = _dot(hj, fcw_ref[...]) + fcb_ref[...]
    m = jnp.max(logits, axis=1, keepdims=True)
    ex = jnp.exp(logits - m)
    lse = jnp.log(jnp.sum(ex, axis=1, keepdims=True)) + m
    out_ref[...] = logits - lse


_f32 = lambda *s: jax.ShapeDtypeStruct(s, jnp.float32)

BN = 2000

_row = lambda w=F: pl.BlockSpec((BN, w), lambda i: (i, 0))
_pair = pl.BlockSpec((2, BN, F), lambda i: (0, i, 0))
_col = pl.BlockSpec((BN, 1), lambda i: (i, 0))
_full = lambda a, b: pl.BlockSpec((a, b), lambda i: (0, 0))

_tc_first = pl.pallas_call(
    _tc_first_body, grid=(N // BN,),
    in_specs=[_row(), _full(F, F), pl.BlockSpec((2, BN, 1), lambda i: (0, i, 0))],
    out_specs=(_row(), _col),
    out_shape=(_f32(N, F), _f32(N, 1)))

_mid_in = [_pair, _row(), _col, _full(1, F), _full(F, F)]
_tc_mid_first = pl.pallas_call(
    _tc_mid_first_body, grid=(N // BN,),
    in_specs=_mid_in, out_specs=(_row(), _row()),
    out_shape=(_f32(N, F), _f32(N, F)))
_tc_mid = pl.pallas_call(
    _tc_mid_body, grid=(N // BN,),
    in_specs=_mid_in + [_row()], out_specs=(_row(), _row()),
    out_shape=(_f32(N, F), _f32(N, F)))
_tc_last = pl.pallas_call(
    _tc_last_body, grid=(N // BN,),
    in_specs=[_pair, _row(), _col, _full(1, F), _row(), _full(F, CLS),
              _full(1, CLS)],
    out_specs=_row(CLS),
    out_shape=_f32(N, CLS))


def kernel(x, edge_index, W0, b0, W1, b1, W2, b2, W3, b3, fcW, fcb):
    E = edge_index.shape[1]
    pad = EPAD - E
    src = jnp.concatenate([edge_index[0].astype(jnp.int32),
                           jnp.zeros((pad,), jnp.int32)])
    dst = jnp.concatenate([edge_index[1].astype(jnp.int32),
                           jnp.full((pad,), N, jnp.int32)])

    degp = _sc_degree(dst)[:, :N, 0:1]
    hws, dinv = _tc_first(x, W0, degp)

    hmax = None
    for i, (b, Wn) in enumerate(((b0, W1), (b1, W2), (b2, W3))):
        p = _sc_aggregate(hws, src, dst)[:, :N, :]
        if i == 0:
            hmax, hws = _tc_mid_first(p, hws, dinv, b.reshape(1, F), Wn)
        else:
            hmax, hws = _tc_mid(p, hws, dinv, b.reshape(1, F), Wn, hmax)

    p = _sc_aggregate(hws, src, dst)[:, :N, :]
    return _tc_last(p, hws, dinv, b3.reshape(1, F), hmax, fcW,
                    fcb.reshape(1, CLS))


# speedup vs baseline: 7.0383x; 1.3675x over previous
import functools

import jax
import jax.numpy as jnp
from jax import lax
from jax.experimental import pallas as pl
from jax.experimental.pallas import tpu as pltpu
from jax.experimental.pallas import tpu_sc as plsc

N = 10000
F = 128
CLS = 64

NC = 2
NS = 16
NW = NC * NS
KE = 64
EPW = 10240
EPAD = NW * EPW
NCHUNK = EPW // KE
NPAD = 10112
STRIPE = NPAD // NS
DEGW = 128
ZR = 8

_mesh = plsc.VectorSubcoreMesh(core_axis_name="c", subcore_axis_name="s")


def _zero_stripe(zbuf, acc, rows0, width):
    @pl.loop(0, ZR)
    def _(r):
        @pl.loop(0, width, step=16)
        def _(c):
            zbuf[r, pl.ds(c, 16)] = jnp.zeros((16,), jnp.float32)

    nfull = (STRIPE // ZR) * ZR

    @pl.loop(0, nfull, step=ZR)
    def _(r):
        pltpu.sync_copy(zbuf, acc.at[pl.ds(rows0 + r, ZR)])

    rem = STRIPE - nfull
    if rem:
        pltpu.sync_copy(zbuf.at[pl.ds(0, rem)], acc.at[pl.ds(rows0 + nfull, rem)])


@functools.partial(
    pl.kernel,
    out_type=jax.ShapeDtypeStruct((NC, NPAD, DEGW), jnp.float32),
    mesh=_mesh,
    scratch_types=[
        pltpu.VMEM((NCHUNK, 2, KE), jnp.int32),
        pltpu.VMEM((KE, DEGW), jnp.float32),
        pltpu.VMEM((ZR, DEGW), jnp.float32),
        pltpu.VMEM_SHARED((NPAD, DEGW), jnp.float32),
    ],
)
def _sc_degree(eidx_hbm, out_hbm, idxv, ones, zbuf, acc):
    cid = lax.axis_index("c")
    sid = lax.axis_index("s")
    wid = sid * NC + cid
    rows0 = sid * STRIPE

    @pl.loop(0, KE)
    def _(r):
        @pl.loop(0, DEGW, step=16)
        def _(c):
            ones[r, pl.ds(c, 16)] = jnp.full((16,), 1.0, jnp.float32)

    _zero_stripe(zbuf, acc, rows0, DEGW)
    pltpu.sync_copy(eidx_hbm.at[wid], idxv)
    plsc.subcore_barrier()

    @pl.loop(0, NCHUNK)
    def _(i):
        pltpu.sync_copy(ones, acc.at[idxv.at[i, 1]], add=True)

    plsc.subcore_barrier()
    pltpu.sync_copy(acc.at[pl.ds(rows0, STRIPE)],
                    out_hbm.at[cid, pl.ds(rows0, STRIPE)])


NBUF = 2
PHASES = 4
CPP = NCHUNK // PHASES


@functools.partial(
    pl.kernel,
    out_type=jax.ShapeDtypeStruct((NC, NPAD, F), jnp.float32),
    mesh=_mesh,
    scratch_types=[
        pltpu.VMEM((CPP, 2, KE), jnp.int32),
        pltpu.VMEM((NBUF * KE, F), jnp.float32),
        pltpu.VMEM((ZR, F), jnp.float32),
        pltpu.VMEM_SHARED((NPAD, F), jnp.float32),
    ] + [pltpu.SemaphoreType.DMA] * NBUF,
)
def _sc_aggregate(hws_hbm, eidx_hbm, out_hbm, idxv, rows, zbuf,
                  acc, *sems):
    cid = lax.axis_index("c")
    sid = lax.axis_index("s")
    wid = sid * NC + cid
    rows0 = sid * STRIPE

    _zero_stripe(zbuf, acc, rows0, F)
    plsc.subcore_barrier()

    def _gather(i, b):
        pltpu.async_copy(hws_hbm.at[idxv.at[i, 0]],
                         rows.at[pl.ds(b * KE, KE)], sems[b])

    def _wait_scatter(i, b):
        pltpu.make_async_copy(hws_hbm.at[idxv.at[i, 0]],
                              rows.at[pl.ds(b * KE, KE)], sems[b]).wait()
        pltpu.sync_copy(rows.at[pl.ds(b * KE, KE)], acc.at[idxv.at[i, 1]],
                        add=True)

    for ph in range(PHASES):
        pltpu.sync_copy(eidx_hbm.at[wid, pl.ds(ph * CPP, CPP)], idxv)

        for b in range(NBUF):
            _gather(b, b)

        @pl.loop(0, CPP - NBUF, step=NBUF)
        def _(i0):
            for b in range(NBUF):
                _wait_scatter(i0 + b, b)
                _gather(i0 + b + NBUF, b)

        for b in range(NBUF):
            _wait_scatter(CPP - NBUF + b, b)

    plsc.subcore_barrier()
    pltpu.sync_copy(acc.at[pl.ds(rows0, STRIPE)],
                    out_hbm.at[cid, pl.ds(rows0, STRIPE)])


def _dot(a, b):
    return jnp.dot(a, b, preferred_element_type=jnp.float32,
                   precision=lax.Precision.HIGHEST)


def _tc_first_body(x_ref, w_ref, degp_ref, hws_ref, dinv_ref):
    deg = degp_ref[0] + degp_ref[1] + 1.0
    dinv = lax.rsqrt(jnp.maximum(deg, 1.0))
    hws_ref[...] = _dot(x_ref[...], w_ref[...]) * dinv
    dinv_ref[...] = dinv


def _tc_mid_first_body(p_ref, hws_ref, dinv_ref, b_ref, w_ref,
                       hmaxo_ref, hwsn_ref):
    dinv = dinv_ref[...]
    pre = p_ref[0] + p_ref[1] + hws_ref[...]
    h = jnp.maximum(pre * dinv + b_ref[...], 0.0)
    hmaxo_ref[...] = h
    hwsn_ref[...] = _dot(h, w_ref[...]) * dinv


def _tc_mid_body(p_ref, hws_ref, dinv_ref, b_ref, w_ref, hmax_ref,
                 hmaxo_ref, hwsn_ref):
    dinv = dinv_ref[...]
    pre = p_ref[0] + p_ref[1] + hws_ref[...]
    h = jnp.maximum(pre * dinv + b_ref[...], 0.0)
    hmaxo_ref[...] = jnp.maximum(hmax_ref[...], h)
    hwsn_ref[...] = _dot(h, w_ref[...]) * dinv


def _tc_last_body(p_ref, hws_ref, dinv_ref, b_ref, hmax_ref, fcw_ref, fcb_ref,
                  out_ref):
    pre = p_ref[0] + p_ref[1] + hws_ref[...]
    h = jnp.maximum(pre * dinv_ref[...] + b_ref[...], 0.0)
    hj = jnp.maximum(hmax_ref[...], h)
    logits = _dot(hj, fcw_ref[...]) + fcb_ref[...]
    m = jnp.max(logits, axis=1, keepdims=True)
    ex = jnp.exp(logits - m)
    lse = jnp.log(jnp.sum(ex, axis=1, keepdims=True)) + m
    out_ref[...] = logits - lse


_f32 = lambda *s: jax.ShapeDtypeStruct(s, jnp.float32)

BN = 2000

_row = lambda w=F: pl.BlockSpec((BN, w), lambda i: (i, 0))
_pair = pl.BlockSpec((2, BN, F), lambda i: (0, i, 0))
_col = pl.BlockSpec((BN, 1), lambda i: (i, 0))
_full = lambda a, b: pl.BlockSpec((a, b), lambda i: (0, 0))

_tc_first = pl.pallas_call(
    _tc_first_body, grid=(N // BN,),
    in_specs=[_row(), _full(F, F), pl.BlockSpec((2, BN, 1), lambda i: (0, i, 0))],
    out_specs=(_row(), _col),
    out_shape=(_f32(N, F), _f32(N, 1)))

_mid_in = [_pair, _row(), _col, _full(1, F), _full(F, F)]
_tc_mid_first = pl.pallas_call(
    _tc_mid_first_body, grid=(N // BN,),
    in_specs=_mid_in, out_specs=(_row(), _row()),
    out_shape=(_f32(N, F), _f32(N, F)))
_tc_mid = pl.pallas_call(
    _tc_mid_body, grid=(N // BN,),
    in_specs=_mid_in + [_row()], out_specs=(_row(), _row()),
    out_shape=(_f32(N, F), _f32(N, F)))
_tc_last = pl.pallas_call(
    _tc_last_body, grid=(N // BN,),
    in_specs=[_pair, _row(), _col, _full(1, F), _row(), _full(F, CLS),
              _full(1, CLS)],
    out_specs=_row(CLS),
    out_shape=_f32(N, CLS))


def kernel(x, edge_index, W0, b0, W1, b1, W2, b2, W3, b3, fcW, fcb):
    E = edge_index.shape[1]
    pad = EPAD - E
    src = jnp.concatenate([edge_index[0].astype(jnp.int32),
                           jnp.zeros((pad,), jnp.int32)]
                          ).reshape(NW, NCHUNK, 1, KE)
    dst = jnp.concatenate([edge_index[1].astype(jnp.int32),
                           jnp.full((pad,), N, jnp.int32)]
                          ).reshape(NW, NCHUNK, 1, KE)
    eidx = jnp.concatenate([src, dst], axis=2)

    degp = _sc_degree(eidx)[:, :N, 0:1]
    hws, dinv = _tc_first(x, W0, degp)

    hmax = None
    for i, (b, Wn) in enumerate(((b0, W1), (b1, W2), (b2, W3))):
        p = _sc_aggregate(hws, eidx)[:, :N, :]
        if i == 0:
            hmax, hws = _tc_mid_first(p, hws, dinv, b.reshape(1, F), Wn)
        else:
            hmax, hws = _tc_mid(p, hws, dinv, b.reshape(1, F), Wn, hmax)

    p = _sc_aggregate(hws, eidx)[:, :N, :]
    return _tc_last(p, hws, dinv, b3.reshape(1, F), hmax, fcW,
                    fcb.reshape(1, CLS))


# speedup vs baseline: 7.0458x; 1.0011x over previous
import functools

import jax
import jax.numpy as jnp
from jax import lax
from jax.experimental import pallas as pl
from jax.experimental.pallas import tpu as pltpu
from jax.experimental.pallas import tpu_sc as plsc

N = 10000
F = 128
CLS = 64

NC = 2
NS = 16
NW = NC * NS
KE = 64
EPW = 10240
EPAD = NW * EPW
NCHUNK = EPW // KE
NPAD = 10112
STRIPE = NPAD // NS
DEGW = 128
ZR = 8

_mesh = plsc.VectorSubcoreMesh(core_axis_name="c", subcore_axis_name="s")


def _zero_stripe(zbuf, acc, rows0, width):
    @pl.loop(0, ZR)
    def _(r):
        @pl.loop(0, width, step=16)
        def _(c):
            zbuf[r, pl.ds(c, 16)] = jnp.zeros((16,), jnp.float32)

    nfull = (STRIPE // ZR) * ZR

    @pl.loop(0, nfull, step=ZR)
    def _(r):
        pltpu.sync_copy(zbuf, acc.at[pl.ds(rows0 + r, ZR)])

    rem = STRIPE - nfull
    if rem:
        pltpu.sync_copy(zbuf.at[pl.ds(0, rem)], acc.at[pl.ds(rows0 + nfull, rem)])


@functools.partial(
    pl.kernel,
    out_type=jax.ShapeDtypeStruct((NC, NPAD, DEGW), jnp.float32),
    mesh=_mesh,
    scratch_types=[
        pltpu.VMEM((NCHUNK, 2, KE), jnp.int32),
        pltpu.VMEM((KE, DEGW), jnp.float32),
        pltpu.VMEM((ZR, DEGW), jnp.float32),
        pltpu.VMEM_SHARED((NPAD, DEGW), jnp.float32),
    ],
)
def _sc_degree(eidx_hbm, out_hbm, idxv, ones, zbuf, acc):
    cid = lax.axis_index("c")
    sid = lax.axis_index("s")
    wid = sid * NC + cid
    rows0 = sid * STRIPE

    @pl.loop(0, KE)
    def _(r):
        @pl.loop(0, DEGW, step=16)
        def _(c):
            ones[r, pl.ds(c, 16)] = jnp.full((16,), 1.0, jnp.float32)

    _zero_stripe(zbuf, acc, rows0, DEGW)
    pltpu.sync_copy(eidx_hbm.at[wid], idxv)
    plsc.subcore_barrier()

    @pl.loop(0, NCHUNK)
    def _(i):
        pltpu.sync_copy(ones, acc.at[idxv.at[i, 1]], add=True)

    plsc.subcore_barrier()
    pltpu.sync_copy(acc.at[pl.ds(rows0, STRIPE)],
                    out_hbm.at[cid, pl.ds(rows0, STRIPE)])


NBUF = 4
PHASES = 4
CPP = NCHUNK // PHASES


@functools.partial(
    pl.kernel,
    out_type=jax.ShapeDtypeStruct((NC, NPAD, F), jnp.float32),
    mesh=_mesh,
    scratch_types=[
        pltpu.VMEM((CPP, 2, KE), jnp.int32),
        pltpu.VMEM((NBUF * KE, F), jnp.float32),
        pltpu.VMEM((ZR, F), jnp.float32),
        pltpu.VMEM_SHARED((NPAD, F), jnp.float32),
    ] + [pltpu.SemaphoreType.DMA] * NBUF,
)
def _sc_aggregate(hws_hbm, eidx_hbm, out_hbm, idxv, rows, zbuf,
                  acc, *sems):
    cid = lax.axis_index("c")
    sid = lax.axis_index("s")
    wid = sid * NC + cid
    rows0 = sid * STRIPE

    _zero_stripe(zbuf, acc, rows0, F)
    plsc.subcore_barrier()

    def _gather(i, b):
        pltpu.async_copy(hws_hbm.at[idxv.at[i, 0]],
                         rows.at[pl.ds(b * KE, KE)], sems[b])

    def _wait_scatter(i, b):
        pltpu.make_async_copy(hws_hbm.at[idxv.at[i, 0]],
                              rows.at[pl.ds(b * KE, KE)], sems[b]).wait()
        pltpu.sync_copy(rows.at[pl.ds(b * KE, KE)], acc.at[idxv.at[i, 1]],
                        add=True)

    for ph in range(PHASES):
        pltpu.sync_copy(eidx_hbm.at[wid, pl.ds(ph * CPP, CPP)], idxv)

        for b in range(NBUF):
            _gather(b, b)

        @pl.loop(0, CPP - NBUF, step=NBUF)
        def _(i0):
            for b in range(NBUF):
                _wait_scatter(i0 + b, b)
                _gather(i0 + b + NBUF, b)

        for b in range(NBUF):
            _wait_scatter(CPP - NBUF + b, b)

    plsc.subcore_barrier()
    pltpu.sync_copy(acc.at[pl.ds(rows0, STRIPE)],
                    out_hbm.at[cid, pl.ds(rows0, STRIPE)])


def _dot(a, b):
    return jnp.dot(a, b, preferred_element_type=jnp.float32,
                   precision=lax.Precision.HIGHEST)


def _tc_first_body(x_ref, w_ref, degp_ref, hws_ref, dinv_ref):
    deg = degp_ref[0] + degp_ref[1] + 1.0
    dinv = lax.rsqrt(jnp.maximum(deg, 1.0))
    hws_ref[...] = _dot(x_ref[...], w_ref[...]) * dinv
    dinv_ref[...] = dinv


def _tc_mid_first_body(p_ref, hws_ref, dinv_ref, b_ref, w_ref,
                       hmaxo_ref, hwsn_ref):
    dinv = dinv_ref[...]
    pre = p_ref[0] + p_ref[1] + hws_ref[...]
    h = jnp.maximum(pre * dinv + b_ref[...], 0.0)
    hmaxo_ref[...] = h
    hwsn_ref[...] = _dot(h, w_ref[...]) * dinv


def _tc_mid_body(p_ref, hws_ref, dinv_ref, b_ref, w_ref, hmax_ref,
                 hmaxo_ref, hwsn_ref):
    dinv = dinv_ref[...]
    pre = p_ref[0] + p_ref[1] + hws_ref[...]
    h = jnp.maximum(pre * dinv + b_ref[...], 0.0)
    hmaxo_ref[...] = jnp.maximum(hmax_ref[...], h)
    hwsn_ref[...] = _dot(h, w_ref[...]) * dinv


def _tc_last_body(p_ref, hws_ref, dinv_ref, b_ref, hmax_ref, fcw_ref, fcb_ref,
                  out_ref):
    pre = p_ref[0] + p_ref[1] + hws_ref[...]
    h = jnp.maximum(pre * dinv_ref[...] + b_ref[...], 0.0)
    hj = jnp.maximum(hmax_ref[...], h)
    logits = _dot(hj, fcw_ref[...]) + fcb_ref[...]
    m = jnp.max(logits, axis=1, keepdims=True)
    ex = jnp.exp(logits - m)
    lse = jnp.log(jnp.sum(ex, axis=1, keepdims=True)) + m
    out_ref[...] = logits - lse


_f32 = lambda *s: jax.ShapeDtypeStruct(s, jnp.float32)

BN = 2000

_row = lambda w=F: pl.BlockSpec((BN, w), lambda i: (i, 0))
_pair = pl.BlockSpec((2, BN, F), lambda i: (0, i, 0))
_col = pl.BlockSpec((BN, 1), lambda i: (i, 0))
_full = lambda a, b: pl.BlockSpec((a, b), lambda i: (0, 0))

_tc_first = pl.pallas_call(
    _tc_first_body, grid=(N // BN,),
    in_specs=[_row(), _full(F, F), pl.BlockSpec((2, BN, 1), lambda i: (0, i, 0))],
    out_specs=(_row(), _col),
    out_shape=(_f32(N, F), _f32(N, 1)))

_mid_in = [_pair, _row(), _col, _full(1, F), _full(F, F)]
_tc_mid_first = pl.pallas_call(
    _tc_mid_first_body, grid=(N // BN,),
    in_specs=_mid_in, out_specs=(_row(), _row()),
    out_shape=(_f32(N, F), _f32(N, F)))
_tc_mid = pl.pallas_call(
    _tc_mid_body, grid=(N // BN,),
    in_specs=_mid_in + [_row()], out_specs=(_row(), _row()),
    out_shape=(_f32(N, F), _f32(N, F)))
_tc_last = pl.pallas_call(
    _tc_last_body, grid=(N // BN,),
    in_specs=[_pair, _row(), _col, _full(1, F), _row(), _full(F, CLS),
              _full(1, CLS)],
    out_specs=_row(CLS),
    out_shape=_f32(N, CLS))


def kernel(x, edge_index, W0, b0, W1, b1, W2, b2, W3, b3, fcW, fcb):
    E = edge_index.shape[1]
    pad = EPAD - E
    src = jnp.concatenate([edge_index[0].astype(jnp.int32),
                           jnp.zeros((pad,), jnp.int32)]
                          ).reshape(NW, NCHUNK, 1, KE)
    dst = jnp.concatenate([edge_index[1].astype(jnp.int32),
                           jnp.full((pad,), N, jnp.int32)]
                          ).reshape(NW, NCHUNK, 1, KE)
    eidx = jnp.concatenate([src, dst], axis=2)

    degp = _sc_degree(eidx)[:, :N, 0:1]
    hws, dinv = _tc_first(x, W0, degp)

    hmax = None
    for i, (b, Wn) in enumerate(((b0, W1), (b1, W2), (b2, W3))):
        p = _sc_aggregate(hws, eidx)[:, :N, :]
        if i == 0:
            hmax, hws = _tc_mid_first(p, hws, dinv, b.reshape(1, F), Wn)
        else:
            hmax, hws = _tc_mid(p, hws, dinv, b.reshape(1, F), Wn, hmax)

    p = _sc_aggregate(hws, eidx)[:, :N, :]
    return _tc_last(p, hws, dinv, b3.reshape(1, F), hmax, fcW,
                    fcb.reshape(1, CLS))


# speedup vs baseline: 7.0805x; 1.0049x over previous
import functools

import jax
import jax.numpy as jnp
from jax import lax
from jax.experimental import pallas as pl
from jax.experimental.pallas import tpu as pltpu
from jax.experimental.pallas import tpu_sc as plsc

N = 10000
F = 128
CLS = 64

NC = 2
NS = 16
NW = NC * NS
KE = 64
EPW = 10240
EPAD = NW * EPW
NCHUNK = EPW // KE
NPAD = 10112
STRIPE = NPAD // NS
DEGW = 128
ZR = 8

_mesh = plsc.VectorSubcoreMesh(core_axis_name="c", subcore_axis_name="s")


def _zero_stripe(zbuf, acc, rows0, width):
    @pl.loop(0, ZR)
    def _(r):
        @pl.loop(0, width, step=16)
        def _(c):
            zbuf[r, pl.ds(c, 16)] = jnp.zeros((16,), jnp.float32)

    nfull = (STRIPE // ZR) * ZR

    @pl.loop(0, nfull, step=ZR)
    def _(r):
        pltpu.sync_copy(zbuf, acc.at[pl.ds(rows0 + r, ZR)])

    rem = STRIPE - nfull
    if rem:
        pltpu.sync_copy(zbuf.at[pl.ds(0, rem)], acc.at[pl.ds(rows0 + nfull, rem)])


@functools.partial(
    pl.kernel,
    out_type=jax.ShapeDtypeStruct((NC, NPAD, DEGW), jnp.float32),
    mesh=_mesh,
    scratch_types=[
        pltpu.VMEM((NCHUNK, 2, KE), jnp.int32),
        pltpu.VMEM((KE, DEGW), jnp.float32),
        pltpu.VMEM((ZR, DEGW), jnp.float32),
        pltpu.VMEM_SHARED((NPAD, DEGW), jnp.float32),
    ],
)
def _sc_degree(eidx_hbm, out_hbm, idxv, ones, zbuf, acc):
    cid = lax.axis_index("c")
    sid = lax.axis_index("s")
    wid = sid * NC + cid
    rows0 = sid * STRIPE

    @pl.loop(0, KE)
    def _(r):
        @pl.loop(0, DEGW, step=16)
        def _(c):
            ones[r, pl.ds(c, 16)] = jnp.full((16,), 1.0, jnp.float32)

    _zero_stripe(zbuf, acc, rows0, DEGW)
    pltpu.sync_copy(eidx_hbm.at[wid], idxv)
    plsc.subcore_barrier()

    @pl.loop(0, NCHUNK)
    def _(i):
        pltpu.sync_copy(ones, acc.at[idxv.at[i, 1]], add=True)

    plsc.subcore_barrier()
    pltpu.sync_copy(acc.at[pl.ds(rows0, STRIPE)],
                    out_hbm.at[cid, pl.ds(rows0, STRIPE)])


NBUF = 4
PHASES = 4
CPP = NCHUNK // PHASES


@functools.partial(
    pl.kernel,
    out_type=jax.ShapeDtypeStruct((NC, NPAD, F), jnp.float32),
    mesh=_mesh,
    scratch_types=[
        pltpu.VMEM((CPP, 2, KE), jnp.int32),
        pltpu.VMEM((NBUF * KE, F), jnp.float32),
        pltpu.VMEM((ZR, F), jnp.float32),
        pltpu.VMEM_SHARED((NPAD, F), jnp.float32),
    ] + [pltpu.SemaphoreType.DMA] * NBUF,
)
def _sc_aggregate(hws_hbm, eidx_hbm, out_hbm, idxv, rows, zbuf,
                  acc, *sems):
    cid = lax.axis_index("c")
    sid = lax.axis_index("s")
    wid = sid * NC + cid
    rows0 = sid * STRIPE

    _zero_stripe(zbuf, acc, rows0, F)
    plsc.subcore_barrier()

    def _gather(i, b):
        pltpu.async_copy(hws_hbm.at[idxv.at[i, 0]],
                         rows.at[pl.ds(b * KE, KE)], sems[b])

    def _wait_scatter(i, b):
        pltpu.make_async_copy(hws_hbm.at[idxv.at[i, 0]],
                              rows.at[pl.ds(b * KE, KE)], sems[b]).wait()
        pltpu.sync_copy(rows.at[pl.ds(b * KE, KE)], acc.at[idxv.at[i, 1]],
                        add=True)

    for ph in range(PHASES):
        pltpu.sync_copy(eidx_hbm.at[wid, pl.ds(ph * CPP, CPP)], idxv)

        for b in range(NBUF):
            _gather(b, b)

        @pl.loop(0, CPP - NBUF, step=NBUF)
        def _(i0):
            for b in range(NBUF):
                _wait_scatter(i0 + b, b)
                _gather(i0 + b + NBUF, b)

        for b in range(NBUF):
            _wait_scatter(CPP - NBUF + b, b)

    plsc.subcore_barrier()
    pltpu.sync_copy(acc.at[pl.ds(rows0, STRIPE)],
                    out_hbm.at[cid, pl.ds(rows0, STRIPE)])


def _dot(a, b):
    return jnp.dot(a, b, preferred_element_type=jnp.float32,
                   precision=lax.Precision.HIGHEST)


def _tc_mm_body(x_ref, w_ref, hw_ref):
    hw_ref[...] = _dot(x_ref[...], w_ref[...])


def _tc_scale_body(hw_ref, degp_ref, hws_ref, dinv_ref):
    deg = degp_ref[0, :, 0:1] + degp_ref[1, :, 0:1] + 1.0
    dinv = lax.rsqrt(jnp.maximum(deg, 1.0))
    hws_ref[...] = hw_ref[...] * dinv
    dinv_ref[...] = dinv


def _tc_mid_first_body(p_ref, hws_ref, dinv_ref, b_ref, w_ref,
                       hmaxo_ref, hwsn_ref):
    dinv = dinv_ref[...]
    pre = p_ref[0] + p_ref[1] + hws_ref[...]
    h = jnp.maximum(pre * dinv + b_ref[...], 0.0)
    hmaxo_ref[...] = h
    hwsn_ref[...] = _dot(h, w_ref[...]) * dinv


def _tc_mid_body(p_ref, hws_ref, dinv_ref, b_ref, w_ref, hmax_ref,
                 hmaxo_ref, hwsn_ref):
    dinv = dinv_ref[...]
    pre = p_ref[0] + p_ref[1] + hws_ref[...]
    h = jnp.maximum(pre * dinv + b_ref[...], 0.0)
    hmaxo_ref[...] = jnp.maximum(hmax_ref[...], h)
    hwsn_ref[...] = _dot(h, w_ref[...]) * dinv


def _tc_last_body(p_ref, hws_ref, dinv_ref, b_ref, hmax_ref, fcw_ref, fcb_ref,
                  out_ref):
    pre = p_ref[0] + p_ref[1] + hws_ref[...]
    h = jnp.maximum(pre * dinv_ref[...] + b_ref[...], 0.0)
    hj = jnp.maximum(hmax_ref[...], h)
    logits = _dot(hj, fcw_ref[...]) + fcb_ref[...]
    m = jnp.max(logits, axis=1, keepdims=True)
    ex = jnp.exp(logits - m)
    lse = jnp.log(jnp.sum(ex, axis=1, keepdims=True)) + m
    out_ref[...] = logits - lse


_f32 = lambda *s: jax.ShapeDtypeStruct(s, jnp.float32)

BN = 2000

_row = lambda w=F: pl.BlockSpec((BN, w), lambda i: (i, 0))
_pair = pl.BlockSpec((2, BN, F), lambda i: (0, i, 0))
_col = pl.BlockSpec((BN, 1), lambda i: (i, 0))
_full = lambda a, b: pl.BlockSpec((a, b), lambda i: (0, 0))

_tc_mm = pl.pallas_call(
    _tc_mm_body, grid=(N // BN,),
    in_specs=[_row(), _full(F, F)],
    out_specs=_row(),
    out_shape=_f32(N, F))

_tc_scale = pl.pallas_call(
    _tc_scale_body, grid=(N // BN,),
    in_specs=[_row(), pl.BlockSpec((2, BN, DEGW), lambda i: (0, i, 0))],
    out_specs=(_row(), _col),
    out_shape=(_f32(N, F), _f32(N, 1)))

_mid_in = [_pair, _row(), _col, _full(1, F), _full(F, F)]
_tc_mid_first = pl.pallas_call(
    _tc_mid_first_body, grid=(N // BN,),
    in_specs=_mid_in, out_specs=(_row(), _row()),
    out_shape=(_f32(N, F), _f32(N, F)))
_tc_mid = pl.pallas_call(
    _tc_mid_body, grid=(N // BN,),
    in_specs=_mid_in + [_row()], out_specs=(_row(), _row()),
    out_shape=(_f32(N, F), _f32(N, F)))
_tc_last = pl.pallas_call(
    _tc_last_body, grid=(N // BN,),
    in_specs=[_pair, _row(), _col, _full(1, F), _row(), _full(F, CLS),
              _full(1, CLS)],
    out_specs=_row(CLS),
    out_shape=_f32(N, CLS))


def kernel(x, edge_index, W0, b0, W1, b1, W2, b2, W3, b3, fcW, fcb):
    E = edge_index.shape[1]
    pad = EPAD - E
    src = jnp.concatenate([edge_index[0].astype(jnp.int32),
                           jnp.zeros((pad,), jnp.int32)]
                          ).reshape(NW, NCHUNK, 1, KE)
    dst = jnp.concatenate([edge_index[1].astype(jnp.int32),
                           jnp.full((pad,), N, jnp.int32)]
                          ).reshape(NW, NCHUNK, 1, KE)
    eidx = jnp.concatenate([src, dst], axis=2)

    degp = _sc_degree(eidx)
    hw0 = _tc_mm(x, W0)
    hws, dinv = _tc_scale(hw0, degp)

    hmax = None
    for i, (b, Wn) in enumerate(((b0, W1), (b1, W2), (b2, W3))):
        p = _sc_aggregate(hws, eidx)
        if i == 0:
            hmax, hws = _tc_mid_first(p, hws, dinv, b.reshape(1, F), Wn)
        else:
            hmax, hws = _tc_mid(p, hws, dinv, b.reshape(1, F), Wn, hmax)

    p = _sc_aggregate(hws, eidx)
    return _tc_last(p, hws, dinv, b3.reshape(1, F), hmax, fcW,
                    fcb.reshape(1, CLS))
